# Initial kernel scaffold; baseline (speedup 1.0000x reference)
import jax
import jax.numpy as jnp
from jax.experimental import pallas as pl


def kernel(x, modality_info, ln_g, ln_b, Wm0, bm0, Wm1, bm1, Wf1, bf1, Wf2, bf2):
    raise NotImplementedError("write your pallas kernel here")


# speedup vs baseline: 1.5404x; 1.5404x over previous
import functools

import jax
import jax.numpy as jnp
from jax.experimental import pallas as pl
from jax.experimental.pallas import tpu as pltpu

B, S, H = 4, 8192, 768
E, M, TOPK = 64, 2, 2
EPM = E // M
N = B * S
T = 512
GRID = N // T


def _router_kernel(x_ref, mi_ref, ln_g_ref, ln_b_ref, wm0_ref, bm0_ref,
                   wm1_ref, bm1_ref, wf1_ref, bf1_ref, wf2_ref, bf2_ref,
                   idx_ref, prob_ref, aux_ref, rpe_acc, mb_acc):
    i = pl.program_id(0)

    @pl.when(i == 0)
    def _init():
        rpe_acc[...] = jnp.zeros_like(rpe_acc)
        mb_acc[...] = jnp.zeros_like(mb_acc)

    h = jnp.dot(mi_ref[...], wf1_ref[...], preferred_element_type=jnp.float32)
    h = jax.nn.relu(h + bf1_ref[...])
    f = jnp.dot(h, wf2_ref[...], preferred_element_type=jnp.float32)
    f = f + bf2_ref[...]
    fmax = jnp.max(f, axis=1, keepdims=True)
    fe = jnp.exp(f - fmax)
    mw = fe / jnp.sum(fe, axis=1, keepdims=True)

    x = x_ref[...]
    mu = jnp.mean(x, axis=1, keepdims=True)
    xc = x - mu
    var = jnp.mean(xc * xc, axis=1, keepdims=True)
    xn = xc * jax.lax.rsqrt(var + 1e-5) * ln_g_ref[...] + ln_b_ref[...]

    l0 = jnp.dot(xn, wm0_ref[...], preferred_element_type=jnp.float32)
    l0 = (l0 + bm0_ref[...]) * mw[:, 0:1]
    l1 = jnp.dot(xn, wm1_ref[...], preferred_element_type=jnp.float32)
    l1 = (l1 + bm1_ref[...]) * mw[:, 1:2]
    logits = jnp.concatenate([l0, l1], axis=1)

    lmax = jnp.max(logits, axis=1, keepdims=True)
    le = jnp.exp(logits - lmax)
    probs = le / jnp.sum(le, axis=1, keepdims=True)

    iota = jax.lax.broadcasted_iota(jnp.int32, (T, E), 1)
    m1 = jnp.max(probs, axis=1, keepdims=True)
    i1 = jnp.min(jnp.where(probs == m1, iota, E), axis=1, keepdims=True)
    probs2 = jnp.where(iota == i1, -1.0, probs)
    m2 = jnp.max(probs2, axis=1, keepdims=True)
    i2 = jnp.min(jnp.where(probs2 == m2, iota, E), axis=1, keepdims=True)
    s = m1 + m2
    idx_ref[...] = jnp.concatenate([i1, i2], axis=1)
    prob_ref[...] = jnp.concatenate([m1 / s, m2 / s], axis=1)

    rpe_acc[...] += jnp.sum(probs, axis=0, keepdims=True)
    mb_acc[...] += jnp.sum(mw, axis=0, keepdims=True)

    @pl.when(i == GRID - 1)
    def _finish():
        rpe = rpe_acc[...] / N
        mb = mb_acc[...] / N
        lb = jnp.sum(rpe * jnp.log(rpe * E + 1e-9), axis=1, keepdims=True)
        ml = jnp.sum(mb * jnp.log(mb * M + 1e-9), axis=1, keepdims=True)
        aux_ref[...] = lb + 0.1 * ml


@functools.partial(jax.jit, static_argnames=("interpret",))
def kernel(x, modality_info, ln_g, ln_b, Wm0, bm0, Wm1, bm1, Wf1, bf1,
           Wf2, bf2, interpret=False):
    x2 = x.reshape(N, H)
    mi2 = modality_info.reshape(N, H * M)
    row = lambda a: a.reshape(1, -1)

    tok_spec = lambda w: pl.BlockSpec((T, w), lambda i: (i, 0))
    full = lambda a: pl.BlockSpec(a.shape, lambda i: (0, 0))

    args = (x2, mi2, row(ln_g), row(ln_b), Wm0, row(bm0), Wm1, row(bm1),
            Wf1, row(bf1), Wf2, row(bf2))
    in_specs = [tok_spec(H), tok_spec(H * M)] + [full(a) for a in args[2:]]

    idx, prob, aux = pl.pallas_call(
        _router_kernel,
        grid=(GRID,),
        in_specs=in_specs,
        out_specs=[
            pl.BlockSpec((T, TOPK), lambda i: (i, 0)),
            pl.BlockSpec((T, TOPK), lambda i: (i, 0)),
            pl.BlockSpec((1, 1), lambda i: (0, 0)),
        ],
        out_shape=[
            jax.ShapeDtypeStruct((N, TOPK), jnp.int32),
            jax.ShapeDtypeStruct((N, TOPK), jnp.float32),
            jax.ShapeDtypeStruct((1, 1), jnp.float32),
        ],
        scratch_shapes=[
            pltpu.VMEM((1, E), jnp.float32),
            pltpu.VMEM((1, M), jnp.float32),
        ],
        compiler_params=pltpu.CompilerParams(
            dimension_semantics=("arbitrary",),
        ),
        interpret=interpret,
    )(*args)

    return (idx.reshape(B, S, TOPK), prob.reshape(B, S, TOPK),
            aux.reshape(()))


# speedup vs baseline: 1.5972x; 1.0368x over previous
import functools

import jax
import jax.numpy as jnp
from jax.experimental import pallas as pl
from jax.experimental.pallas import tpu as pltpu

B, S, H = 4, 8192, 768
E, M, TOPK = 64, 2, 2
EPM = E // M
N = B * S
T = 1024
GRID = N // T


def _router_kernel(x_ref, mi_ref, ln_g_ref, ln_b_ref, wm_ref, bm_ref,
                   wf1_ref, bf1_ref, wf2_ref, bf2_ref,
                   idx_ref, prob_ref, aux_ref, rpe_acc, mb_acc):
    i = pl.program_id(0)

    @pl.when(i == 0)
    def _init():
        rpe_acc[...] = jnp.zeros_like(rpe_acc)
        mb_acc[...] = jnp.zeros_like(mb_acc)

    h = jnp.dot(mi_ref[...], wf1_ref[...], preferred_element_type=jnp.float32)
    h = jax.nn.relu(h + bf1_ref[...])
    f = jnp.dot(h, wf2_ref[...], preferred_element_type=jnp.float32)
    f = f + bf2_ref[...]
    fmax = jnp.max(f, axis=1, keepdims=True)
    fe = jnp.exp(f - fmax)
    mw = fe / jnp.sum(fe, axis=1, keepdims=True)

    x = x_ref[...]
    mu = jnp.mean(x, axis=1, keepdims=True)
    xc = x - mu
    var = jnp.mean(xc * xc, axis=1, keepdims=True)
    xn = xc * jax.lax.rsqrt(var + 1e-5) * ln_g_ref[...] + ln_b_ref[...]

    iota = jax.lax.broadcasted_iota(jnp.int32, (T, E), 1)
    lm = jnp.dot(xn, wm_ref[...], preferred_element_type=jnp.float32)
    scale = jnp.where(iota < EPM, mw[:, 0:1], mw[:, 1:2])
    logits = (lm + bm_ref[...]) * scale

    lmax = jnp.max(logits, axis=1, keepdims=True)
    le = jnp.exp(logits - lmax)
    probs = le / jnp.sum(le, axis=1, keepdims=True)

    m1 = jnp.max(probs, axis=1, keepdims=True)
    i1 = jnp.min(jnp.where(probs == m1, iota, E), axis=1, keepdims=True)
    probs2 = jnp.where(iota == i1, -1.0, probs)
    m2 = jnp.max(probs2, axis=1, keepdims=True)
    i2 = jnp.min(jnp.where(probs2 == m2, iota, E), axis=1, keepdims=True)
    s = m1 + m2
    idx_ref[...] = jnp.concatenate([i1, i2], axis=1)
    prob_ref[...] = jnp.concatenate([m1 / s, m2 / s], axis=1)

    rpe_acc[...] += jnp.sum(probs, axis=0, keepdims=True)
    mb_acc[...] += jnp.sum(mw, axis=0, keepdims=True)

    @pl.when(i == GRID - 1)
    def _finish():
        rpe = rpe_acc[...] / N
        mb = mb_acc[...] / N
        lb = jnp.sum(rpe * jnp.log(rpe * E + 1e-9), axis=1, keepdims=True)
        ml = jnp.sum(mb * jnp.log(mb * M + 1e-9), axis=1, keepdims=True)
        aux_ref[...] = lb + 0.1 * ml


@functools.partial(jax.jit, static_argnames=("interpret",))
def kernel(x, modality_info, ln_g, ln_b, Wm0, bm0, Wm1, bm1, Wf1, bf1,
           Wf2, bf2, interpret=False):
    x2 = x.reshape(N, H)
    mi2 = modality_info.reshape(N, H * M)
    row = lambda a: a.reshape(1, -1)
    Wm = jnp.concatenate([Wm0, Wm1], axis=1)
    bm = jnp.concatenate([bm0, bm1]).reshape(1, E)

    tok_spec = lambda w: pl.BlockSpec((T, w), lambda i: (i, 0))
    full = lambda a: pl.BlockSpec(a.shape, lambda i: (0, 0))

    args = (x2, mi2, row(ln_g), row(ln_b), Wm, bm,
            Wf1, row(bf1), Wf2, row(bf2))
    in_specs = [tok_spec(H), tok_spec(H * M)] + [full(a) for a in args[2:]]

    idx, prob, aux = pl.pallas_call(
        _router_kernel,
        grid=(GRID,),
        in_specs=in_specs,
        out_specs=[
            pl.BlockSpec((T, TOPK), lambda i: (i, 0)),
            pl.BlockSpec((T, TOPK), lambda i: (i, 0)),
            pl.BlockSpec((1, 1), lambda i: (0, 0)),
        ],
        out_shape=[
            jax.ShapeDtypeStruct((N, TOPK), jnp.int32),
            jax.ShapeDtypeStruct((N, TOPK), jnp.float32),
            jax.ShapeDtypeStruct((1, 1), jnp.float32),
        ],
        scratch_shapes=[
            pltpu.VMEM((1, E), jnp.float32),
            pltpu.VMEM((1, M), jnp.float32),
        ],
        compiler_params=pltpu.CompilerParams(
            dimension_semantics=("arbitrary",),
        ),
        interpret=interpret,
    )(*args)

    return (idx.reshape(B, S, TOPK), prob.reshape(B, S, TOPK),
            aux.reshape(()))
